# Initial kernel scaffold; baseline (speedup 1.0000x reference)
import jax
import jax.numpy as jnp
from jax.experimental import pallas as pl


def kernel(X, params):
    raise NotImplementedError("write your pallas kernel here")


# speedup vs baseline: 2.1929x; 2.1929x over previous
import numpy as np
import jax
import jax.numpy as jnp
from jax.experimental import pallas as pl
from jax.experimental.pallas import tpu as pltpu

N_STEPS = 4
TB = 8
B = 256
NV = 64
NE = 1152


def _build_graph():
    s = set()
    for i in range(8):
        for j in range(8):
            start = 8 * i + j
            for x in range(8):
                s.add((start, 8 * i + x))
                s.add((start, 8 * x + j))
            bx = i // 2 * 2
            by = j // 4 * 4
            for x in range(2):
                for y in range(4):
                    s.add((start, 8 * (bx + x) + (by + y)))
    pairs = sorted(s)
    l = np.array([p[0] for p in pairs], dtype=np.int32)
    r = np.array([p[1] for p in pairs], dtype=np.int32)
    return l, r


_L, _R = _build_graph()

_GL = np.zeros((NE, NV), np.float32)
_GL[np.arange(NE), _L] = 1.0
_GR = np.zeros((NE, NV), np.float32)
_GR[np.arange(NE), _R] = 1.0
_S = np.zeros((NV, NE), np.float32)
_S[_L, np.arange(NE)] = 1.0

_t = np.eye(8, dtype=np.float32)
_RC = np.concatenate([np.tile(_t, (8, 1)), np.repeat(_t, 8, axis=0)], axis=1)


def _mlp(x, w):
    for i in range(3):
        x = jnp.maximum(
            jnp.dot(x, w[2 * i], preferred_element_type=jnp.float32) + w[2 * i + 1],
            0.0)
    return jnp.dot(x, w[6], preferred_element_type=jnp.float32) + w[7]


def _body(x_ref, *refs):
    out_ref = refs[-1]
    w = [r[...] for r in refs[:-1]]
    (inp_w, msg_w, comb_w) = (w[0:8], w[8:16], w[16:24])
    wih, bih, whh, bhh, wd, bd, gl, gr, sm = w[24:]

    x = x_ref[...]
    xe = _mlp(x, inp_w)
    h = xe
    c = jnp.zeros_like(h)

    for _ in range(N_STEPS):
        es = []
        for s in range(TB):
            hs = h[s * NV:(s + 1) * NV]
            es.append(jnp.concatenate(
                [jnp.dot(gl, hs, preferred_element_type=jnp.float32),
                 jnp.dot(gr, hs, preferred_element_type=jnp.float32)], axis=1))
        e = jnp.concatenate(es, axis=0)
        msg = _mlp(e, msg_w)
        aggs = [jnp.dot(sm, msg[s * NE:(s + 1) * NE],
                        preferred_element_type=jnp.float32)
                for s in range(TB)]
        agg = jnp.concatenate(aggs, axis=0)
        xm = _mlp(jnp.concatenate([xe, agg], axis=1), comb_w)
        gates = (jnp.dot(xm, wih, preferred_element_type=jnp.float32) + bih
                 + jnp.dot(h, whh, preferred_element_type=jnp.float32) + bhh)
        i_g = gates[:, 0:16]
        f_g = gates[:, 16:32]
        g_g = gates[:, 32:48]
        o_g = gates[:, 48:64]
        c = jax.nn.sigmoid(f_g) * c + jax.nn.sigmoid(i_g) * jnp.tanh(g_g)
        h = jax.nn.sigmoid(o_g) * jnp.tanh(c)

    out_ref[...] = jnp.dot(h, wd, preferred_element_type=jnp.float32) + bd


def _mlp_weights(p):
    out = []
    for i in range(4):
        out.append(p[f"W{i}"].T)
        out.append(p[f"b{i}"].reshape(1, -1))
    return out


def kernel(X, params):
    Xf = X.reshape(B * NV, 9).astype(jnp.float32)
    rc = jnp.asarray(np.tile(_RC, (B, 1)))
    xin = jnp.concatenate([rc, Xf], axis=1)

    weights = (
        _mlp_weights(params["inp_enc"])
        + _mlp_weights(params["msg_enc"])
        + _mlp_weights(params["msg_comb"])
        + [params["W_ih"].T, params["b_ih"].reshape(1, -1),
           params["W_hh"].T, params["b_hh"].reshape(1, -1),
           params["Wd"].T, params["bd"].reshape(1, -1),
           jnp.asarray(_GL), jnp.asarray(_GR), jnp.asarray(_S)]
    )

    grid = (B // TB,)
    in_specs = [pl.BlockSpec((TB * NV, 25), lambda i: (i, 0))]
    for a in weights:
        in_specs.append(pl.BlockSpec(a.shape, lambda i: (0,) * a.ndim))

    out = pl.pallas_call(
        _body,
        grid=grid,
        in_specs=in_specs,
        out_specs=pl.BlockSpec((TB * NV, 8), lambda i: (i, 0)),
        out_shape=jax.ShapeDtypeStruct((B * NV, 8), jnp.float32),
        compiler_params=pltpu.CompilerParams(
            dimension_semantics=("arbitrary",)),
    )(xin, *weights)
    return out


# speedup vs baseline: 2.5450x; 1.1606x over previous
import numpy as np
import jax
import jax.numpy as jnp
from jax.experimental import pallas as pl
from jax.experimental.pallas import tpu as pltpu

N_STEPS = 4
TB = 8
B = 256
NV = 64
NE = 1152
DEG = 18


def _build_graph():
    s = set()
    for i in range(8):
        for j in range(8):
            start = 8 * i + j
            for x in range(8):
                s.add((start, 8 * i + x))
                s.add((start, 8 * x + j))
            bx = i // 2 * 2
            by = j // 4 * 4
            for x in range(2):
                for y in range(4):
                    s.add((start, 8 * (bx + x) + (by + y)))
    pairs = sorted(s)
    l = np.array([p[0] for p in pairs], dtype=np.int32)
    r = np.array([p[1] for p in pairs], dtype=np.int32)
    return l, r


_L, _R = _build_graph()

_GL = np.zeros((NE, NV), np.float32)
_GL[np.arange(NE), _L] = 1.0
_GR = np.zeros((NE, NV), np.float32)
_GR[np.arange(NE), _R] = 1.0
_S = np.zeros((NV, NE), np.float32)
_S[_L, np.arange(NE)] = 1.0

_t = np.eye(8, dtype=np.float32)
_RC = np.concatenate([np.tile(_t, (8, 1)), np.repeat(_t, 8, axis=0)], axis=1)

_BF = jnp.bfloat16


def _dot(a, b):
    return jnp.dot(a, b, preferred_element_type=jnp.float32)


def _mlp(x_bf, w):
    for i in range(3):
        z = _dot(x_bf, w[2 * i]) + w[2 * i + 1]
        x_bf = jnp.maximum(z, 0.0).astype(_BF)
    return _dot(x_bf, w[6]) + w[7]


def _body(x_ref, *refs):
    out_ref = refs[-1]
    w = [r[...] for r in refs[:-1]]
    inp_w, comb_w = w[0:8], w[8:16]
    (w0l, w0r, b0m, w1m, b1m, w2m, b2m, w3m, b3m,
     wih, bih, whh, bhh, wd, bd, gl, gr, sm) = w[16:]

    x = x_ref[...].astype(_BF)
    xe = _mlp(x, inp_w)
    h = xe
    c = jnp.zeros_like(h)

    for _ in range(N_STEPS):
        h_bf = h.astype(_BF)
        ps = []
        for s in range(TB):
            hs = h_bf[s * NV:(s + 1) * NV]
            u = _dot(hs, w0l).astype(_BF)
            v = _dot(hs, w0r).astype(_BF)
            ps.append(_dot(gl, u) + _dot(gr, v))
        z = jnp.concatenate(ps, axis=0) + b0m
        z = jnp.maximum(z, 0.0).astype(_BF)
        z = jnp.maximum(_dot(z, w1m) + b1m, 0.0).astype(_BF)
        z = jnp.maximum(_dot(z, w2m) + b2m, 0.0).astype(_BF)
        rs = [_dot(sm, z[s * NE:(s + 1) * NE]).astype(_BF) for s in range(TB)]
        red = jnp.concatenate(rs, axis=0)
        agg = _dot(red, w3m) + b3m
        xm = _mlp(jnp.concatenate([xe, agg], axis=1).astype(_BF), comb_w)
        gates = (_dot(xm.astype(_BF), wih) + bih
                 + _dot(h_bf, whh) + bhh)
        i_g = gates[:, 0:16]
        f_g = gates[:, 16:32]
        g_g = gates[:, 32:48]
        o_g = gates[:, 48:64]
        c = jax.nn.sigmoid(f_g) * c + jax.nn.sigmoid(i_g) * jnp.tanh(g_g)
        h = jax.nn.sigmoid(o_g) * jnp.tanh(c)

    out_ref[...] = _dot(h.astype(_BF), wd) + bd


def _mlp_weights(p):
    out = []
    for i in range(4):
        out.append(p[f"W{i}"].T.astype(_BF))
        out.append(p[f"b{i}"].reshape(1, -1))
    return out


def kernel(X, params):
    Xf = X.reshape(B * NV, 9).astype(jnp.float32)
    rc = jnp.asarray(np.tile(_RC, (B, 1)))
    xin = jnp.concatenate([rc, Xf], axis=1)

    pm = params["msg_enc"]
    msg_w = [
        pm["W0"][:, :16].T.astype(_BF),
        pm["W0"][:, 16:].T.astype(_BF),
        pm["b0"].reshape(1, -1),
        pm["W1"].T.astype(_BF), pm["b1"].reshape(1, -1),
        pm["W2"].T.astype(_BF), pm["b2"].reshape(1, -1),
        pm["W3"].T.astype(_BF),
        (DEG * pm["b3"]).reshape(1, -1),
    ]

    weights = (
        _mlp_weights(params["inp_enc"])
        + _mlp_weights(params["msg_comb"])
        + msg_w
        + [params["W_ih"].T.astype(_BF), params["b_ih"].reshape(1, -1),
           params["W_hh"].T.astype(_BF), params["b_hh"].reshape(1, -1),
           params["Wd"].T.astype(_BF), params["bd"].reshape(1, -1),
           jnp.asarray(_GL, _BF), jnp.asarray(_GR, _BF), jnp.asarray(_S, _BF)]
    )

    grid = (B // TB,)
    in_specs = [pl.BlockSpec((TB * NV, 25), lambda i: (i, 0))]
    for a in weights:
        in_specs.append(pl.BlockSpec(a.shape, lambda i: (0,) * a.ndim))

    out = pl.pallas_call(
        _body,
        grid=grid,
        in_specs=in_specs,
        out_specs=pl.BlockSpec((TB * NV, 8), lambda i: (i, 0)),
        out_shape=jax.ShapeDtypeStruct((B * NV, 8), jnp.float32),
        compiler_params=pltpu.CompilerParams(
            dimension_semantics=("arbitrary",)),
    )(xin, *weights)
    return out


# speedup vs baseline: 3.3772x; 1.3270x over previous
import numpy as np
import jax
import jax.numpy as jnp
from jax.experimental import pallas as pl
from jax.experimental.pallas import tpu as pltpu

N_STEPS = 4
TB = 8
B = 256
NV = 64
NE = 1152
DEG = 18


def _build_graph():
    s = set()
    for i in range(8):
        for j in range(8):
            start = 8 * i + j
            for x in range(8):
                s.add((start, 8 * i + x))
                s.add((start, 8 * x + j))
            bx = i // 2 * 2
            by = j // 4 * 4
            for x in range(2):
                for y in range(4):
                    s.add((start, 8 * (bx + x) + (by + y)))
    pairs = sorted(s)
    l = np.array([p[0] for p in pairs], dtype=np.int32)
    r = np.array([p[1] for p in pairs], dtype=np.int32)
    return l, r


_L, _R = _build_graph()

_perm = np.argsort(np.arange(NE) % DEG, kind="stable")
_Lp, _Rp = _L[_perm], _R[_perm]

_GL = np.zeros((NE, NV), np.float32)
_GL[np.arange(NE), _Lp] = 1.0
_GR = np.zeros((NE, NV), np.float32)
_GR[np.arange(NE), _Rp] = 1.0

_G2 = np.concatenate([_GL, _GR], axis=1)

_t = np.eye(8, dtype=np.float32)
_RC = np.concatenate([np.tile(_t, (8, 1)), np.repeat(_t, 8, axis=0)], axis=1)

_BF = jnp.bfloat16


def _dot(a, b):
    return jnp.dot(a, b, preferred_element_type=jnp.float32)


def _dotb(a, b):
    return jnp.dot(a, b, preferred_element_type=jnp.float32).astype(_BF)


def _mlp(x_bf, w):
    for i in range(3):
        x_bf = jnp.maximum(_dotb(x_bf, w[2 * i]) + w[2 * i + 1], 0.0)
    return _dot(x_bf, w[6]) + w[7]


def _body(x_ref, *refs):
    out_ref = refs[-1]
    w = [r[...] for r in refs[:-1]]
    inp_w, comb_w = w[0:8], w[8:16]
    (w0l, w0r, b0m, w1m, b1m, w2m, b2m, w3m, b3m,
     wih, bih, whh, bhh, wd, bd, g2) = w[16:]

    x = x_ref[...].astype(_BF)
    xe = _mlp(x, inp_w)
    h = xe
    c = jnp.zeros_like(h)

    for _ in range(N_STEPS):
        h_bf = h.astype(_BF)
        u = _dotb(h_bf, w0l)
        v = _dotb(h_bf, w0r)
        rs = []
        for p in range(TB // 2):
            zp = []
            for s in (2 * p, 2 * p + 1):
                uv = jnp.concatenate(
                    [u[s * NV:(s + 1) * NV], v[s * NV:(s + 1) * NV]], axis=0)
                zp.append(_dotb(g2, uv))
            z = jnp.concatenate(zp, axis=0)
            z = jnp.maximum(z + b0m, 0.0)
            z = jnp.maximum(_dotb(z, w1m) + b1m, 0.0)
            z = jnp.maximum(_dotb(z, w2m) + b2m, 0.0)
            for si in range(2):
                zs = z[si * NE:(si + 1) * NE]
                blocks = [zs[k * NV:(k + 1) * NV] for k in range(DEG)]
                while len(blocks) > 1:
                    nb = [blocks[j] + blocks[j + 1]
                          for j in range(0, len(blocks) - 1, 2)]
                    if len(blocks) % 2:
                        nb.append(blocks[-1])
                    blocks = nb
                rs.append(blocks[0])
        red = jnp.concatenate(rs, axis=0)
        agg = _dot(red, w3m) + b3m
        xm = _mlp(jnp.concatenate([xe, agg], axis=1).astype(_BF), comb_w)
        gates = (_dot(xm.astype(_BF), wih) + bih
                 + _dot(h_bf, whh) + bhh)
        i_g = gates[:, 0:16]
        f_g = gates[:, 16:32]
        g_g = gates[:, 32:48]
        o_g = gates[:, 48:64]
        c = jax.nn.sigmoid(f_g) * c + jax.nn.sigmoid(i_g) * jnp.tanh(g_g)
        h = jax.nn.sigmoid(o_g) * jnp.tanh(c)

    out_ref[...] = _dot(h.astype(_BF), wd) + bd


def _mlp_weights(p):
    out = []
    for i in range(4):
        out.append(p[f"W{i}"].T.astype(_BF))
        b = p[f"b{i}"].reshape(1, -1)
        out.append(b if i == 3 else b.astype(_BF))
    return out


def kernel(X, params):
    Xf = X.reshape(B * NV, 9).astype(jnp.float32)
    rc = jnp.asarray(np.tile(_RC, (B, 1)))
    xin = jnp.concatenate([rc, Xf], axis=1)

    pm = params["msg_enc"]
    msg_w = [
        pm["W0"][:, :16].T.astype(_BF),
        pm["W0"][:, 16:].T.astype(_BF),
        pm["b0"].reshape(1, -1).astype(_BF),
        pm["W1"].T.astype(_BF), pm["b1"].reshape(1, -1).astype(_BF),
        pm["W2"].T.astype(_BF), pm["b2"].reshape(1, -1).astype(_BF),
        pm["W3"].T.astype(_BF),
        (DEG * pm["b3"]).reshape(1, -1),
    ]

    weights = (
        _mlp_weights(params["inp_enc"])
        + _mlp_weights(params["msg_comb"])
        + msg_w
        + [params["W_ih"].T.astype(_BF), params["b_ih"].reshape(1, -1),
           params["W_hh"].T.astype(_BF), params["b_hh"].reshape(1, -1),
           params["Wd"].T.astype(_BF), params["bd"].reshape(1, -1),
           jnp.asarray(_G2, _BF)]
    )

    grid = (B // TB,)
    in_specs = [pl.BlockSpec((TB * NV, 25), lambda i: (i, 0))]
    for a in weights:
        in_specs.append(pl.BlockSpec(a.shape, lambda i: (0,) * a.ndim))

    out = pl.pallas_call(
        _body,
        grid=grid,
        in_specs=in_specs,
        out_specs=pl.BlockSpec((TB * NV, 8), lambda i: (i, 0)),
        out_shape=jax.ShapeDtypeStruct((B * NV, 8), jnp.float32),
        compiler_params=pltpu.CompilerParams(
            dimension_semantics=("arbitrary",)),
    )(xin, *weights)
    return out


# speedup vs baseline: 3.6514x; 1.0812x over previous
import numpy as np
import jax
import jax.numpy as jnp
from jax.experimental import pallas as pl
from jax.experimental.pallas import tpu as pltpu

N_STEPS = 4
TB = 32
B = 256
NV = 64
NE = 1152
DEG = 18


def _build_graph():
    s = set()
    for i in range(8):
        for j in range(8):
            start = 8 * i + j
            for x in range(8):
                s.add((start, 8 * i + x))
                s.add((start, 8 * x + j))
            bx = i // 2 * 2
            by = j // 4 * 4
            for x in range(2):
                for y in range(4):
                    s.add((start, 8 * (bx + x) + (by + y)))
    pairs = sorted(s)
    l = np.array([p[0] for p in pairs], dtype=np.int32)
    r = np.array([p[1] for p in pairs], dtype=np.int32)
    return l, r


_L, _R = _build_graph()

_perm = np.argsort(np.arange(NE) % DEG, kind="stable")
_Lp, _Rp = _L[_perm], _R[_perm]

_GL = np.zeros((NE, NV), np.float32)
_GL[np.arange(NE), _Lp] = 1.0
_GR = np.zeros((NE, NV), np.float32)
_GR[np.arange(NE), _Rp] = 1.0

_G2 = np.concatenate([_GL, _GR], axis=1)

_t = np.eye(8, dtype=np.float32)
_RC = np.concatenate([np.tile(_t, (8, 1)), np.repeat(_t, 8, axis=0)], axis=1)

_BF = jnp.bfloat16


def _dot(a, b):
    return jnp.dot(a, b, preferred_element_type=jnp.float32)


def _dotb(a, b):
    return jnp.dot(a, b, preferred_element_type=jnp.float32).astype(_BF)


def _mlp(x_bf, w):
    for i in range(3):
        x_bf = jnp.maximum(_dotb(x_bf, w[2 * i]) + w[2 * i + 1], 0.0)
    return _dot(x_bf, w[6]) + w[7]


def _body(x_ref, *refs):
    out_ref = refs[-1]
    w = [r[...] for r in refs[:-1]]
    inp_w, comb_w = w[0:8], w[8:16]
    (w0l, w0r, b0m, w1m, b1m, w2m, b2m, w3m, b3m,
     wih, bih, whh, bhh, wd, bd, g2) = w[16:]

    x = x_ref[...].astype(_BF)
    xe = _mlp(x, inp_w)
    h = xe
    c = jnp.zeros_like(h)

    for _ in range(N_STEPS):
        h_bf = h.astype(_BF)
        u = _dotb(h_bf, w0l)
        v = _dotb(h_bf, w0r)
        rs = []
        for p in range(TB // 2):
            zp = []
            for s in (2 * p, 2 * p + 1):
                uv = jnp.concatenate(
                    [u[s * NV:(s + 1) * NV], v[s * NV:(s + 1) * NV]], axis=0)
                zp.append(_dotb(g2, uv))
            z = jnp.concatenate(zp, axis=0)
            z = jnp.maximum(z + b0m, 0.0)
            z = jnp.maximum(_dotb(z, w1m) + b1m, 0.0)
            z = jnp.maximum(_dotb(z, w2m) + b2m, 0.0)
            for si in range(2):
                zs = z[si * NE:(si + 1) * NE]
                blocks = [zs[k * NV:(k + 1) * NV] for k in range(DEG)]
                while len(blocks) > 1:
                    nb = [blocks[j] + blocks[j + 1]
                          for j in range(0, len(blocks) - 1, 2)]
                    if len(blocks) % 2:
                        nb.append(blocks[-1])
                    blocks = nb
                rs.append(blocks[0])
        red = jnp.concatenate(rs, axis=0)
        agg = _dot(red, w3m) + b3m
        xm = _mlp(jnp.concatenate([xe, agg], axis=1).astype(_BF), comb_w)
        gates = (_dot(xm.astype(_BF), wih) + bih
                 + _dot(h_bf, whh) + bhh)
        i_g = gates[:, 0:16]
        f_g = gates[:, 16:32]
        g_g = gates[:, 32:48]
        o_g = gates[:, 48:64]
        c = jax.nn.sigmoid(f_g) * c + jax.nn.sigmoid(i_g) * jnp.tanh(g_g)
        h = jax.nn.sigmoid(o_g) * jnp.tanh(c)

    out_ref[...] = _dot(h.astype(_BF), wd) + bd


def _mlp_weights(p):
    out = []
    for i in range(4):
        out.append(p[f"W{i}"].T.astype(_BF))
        b = p[f"b{i}"].reshape(1, -1)
        out.append(b if i == 3 else b.astype(_BF))
    return out


def kernel(X, params):
    Xf = X.reshape(B * NV, 9).astype(jnp.float32)
    rc = jnp.asarray(np.tile(_RC, (B, 1)))
    xin = jnp.concatenate([rc, Xf], axis=1)

    pm = params["msg_enc"]
    msg_w = [
        pm["W0"][:, :16].T.astype(_BF),
        pm["W0"][:, 16:].T.astype(_BF),
        pm["b0"].reshape(1, -1).astype(_BF),
        pm["W1"].T.astype(_BF), pm["b1"].reshape(1, -1).astype(_BF),
        pm["W2"].T.astype(_BF), pm["b2"].reshape(1, -1).astype(_BF),
        pm["W3"].T.astype(_BF),
        (DEG * pm["b3"]).reshape(1, -1),
    ]

    weights = (
        _mlp_weights(params["inp_enc"])
        + _mlp_weights(params["msg_comb"])
        + msg_w
        + [params["W_ih"].T.astype(_BF), params["b_ih"].reshape(1, -1),
           params["W_hh"].T.astype(_BF), params["b_hh"].reshape(1, -1),
           params["Wd"].T.astype(_BF), params["bd"].reshape(1, -1),
           jnp.asarray(_G2, _BF)]
    )

    grid = (B // TB,)
    in_specs = [pl.BlockSpec((TB * NV, 25), lambda i: (i, 0))]
    for a in weights:
        in_specs.append(pl.BlockSpec(a.shape, lambda i: (0,) * a.ndim))

    out = pl.pallas_call(
        _body,
        grid=grid,
        in_specs=in_specs,
        out_specs=pl.BlockSpec((TB * NV, 8), lambda i: (i, 0)),
        out_shape=jax.ShapeDtypeStruct((B * NV, 8), jnp.float32),
        compiler_params=pltpu.CompilerParams(
            dimension_semantics=("arbitrary",)),
    )(xin, *weights)
    return out


# speedup vs baseline: 3.8171x; 1.0454x over previous
import numpy as np
import jax
import jax.numpy as jnp
from jax.experimental import pallas as pl
from jax.experimental.pallas import tpu as pltpu

N_STEPS = 4
TB = 32
B = 256
NV = 64
NE = 1152
DEG = 18


def _build_graph():
    s = set()
    for i in range(8):
        for j in range(8):
            start = 8 * i + j
            for x in range(8):
                s.add((start, 8 * i + x))
                s.add((start, 8 * x + j))
            bx = i // 2 * 2
            by = j // 4 * 4
            for x in range(2):
                for y in range(4):
                    s.add((start, 8 * (bx + x) + (by + y)))
    pairs = sorted(s)
    l = np.array([p[0] for p in pairs], dtype=np.int32)
    r = np.array([p[1] for p in pairs], dtype=np.int32)
    return l, r


_L, _R = _build_graph()

_perm = np.argsort(np.arange(NE) % DEG, kind="stable")
_Lp, _Rp = _L[_perm], _R[_perm]

_GL = np.zeros((NE, NV), np.float32)
_GL[np.arange(NE), _Lp] = 1.0
_GR = np.zeros((NE, NV), np.float32)
_GR[np.arange(NE), _Rp] = 1.0

_G2 = np.concatenate([_GL, _GR], axis=1)

_t = np.eye(8, dtype=np.float32)
_RC = np.concatenate([np.tile(_t, (8, 1)), np.repeat(_t, 8, axis=0)], axis=1)

_BF = jnp.bfloat16


def _dot(a, b):
    return jnp.dot(a, b, preferred_element_type=jnp.float32)


def _dotb(a, b):
    return jnp.dot(a, b, preferred_element_type=jnp.float32).astype(_BF)


def _dott(a, b):
    return jax.lax.dot_general(a, b, (((1,), (1,)), ((), ())),
                               preferred_element_type=jnp.float32)


def _dottb(a, b):
    return _dott(a, b).astype(_BF)


def _mlp(x_bf, w):
    for i in range(3):
        x_bf = jnp.maximum(
            _dottb(x_bf, w[2 * i]) + w[2 * i + 1].astype(_BF), 0.0)
    return _dott(x_bf, w[6]) + w[7]


def _body(x_ref, *refs):
    out_ref = refs[-1]
    raw = [r[...] for r in refs[:-1]]
    w = [a.astype(_BF) if (a.ndim == 2 and a.shape[0] > 1) else a
         for a in raw]
    inp_w, comb_w = w[0:8], w[8:16]
    (w0m, b0m, w1m, b1m, w2m, b2m, w3m, b3r,
     wih, bih, whh, bhh, wd, bd, g2) = w[16:]
    w0l = w0m[:, :16]
    w0r = w0m[:, 16:]
    b0m = b0m.astype(_BF)
    b1m = b1m.astype(_BF)
    b2m = b2m.astype(_BF)
    b3m = b3r * np.float32(DEG)

    x = x_ref[...].astype(_BF)
    xe = _mlp(x, inp_w)
    h = xe
    c = jnp.zeros_like(h)

    for _ in range(N_STEPS):
        h_bf = h.astype(_BF)
        u = _dottb(h_bf, w0l)
        v = _dottb(h_bf, w0r)
        rs = []
        for p in range(TB // 2):
            zp = []
            for s in (2 * p, 2 * p + 1):
                uv = jnp.concatenate(
                    [u[s * NV:(s + 1) * NV], v[s * NV:(s + 1) * NV]], axis=0)
                zp.append(_dotb(g2, uv))
            z = jnp.concatenate(zp, axis=0)
            z = jnp.maximum(z + b0m, 0.0)
            z = jnp.maximum(_dottb(z, w1m) + b1m, 0.0)
            z = jnp.maximum(_dottb(z, w2m) + b2m, 0.0)
            for si in range(2):
                zs = z[si * NE:(si + 1) * NE]
                blocks = [zs[k * NV:(k + 1) * NV] for k in range(DEG)]
                while len(blocks) > 1:
                    nb = [blocks[j] + blocks[j + 1]
                          for j in range(0, len(blocks) - 1, 2)]
                    if len(blocks) % 2:
                        nb.append(blocks[-1])
                    blocks = nb
                rs.append(blocks[0])
        red = jnp.concatenate(rs, axis=0)
        agg = _dott(red, w3m) + b3m
        xm = _mlp(jnp.concatenate([xe, agg], axis=1).astype(_BF), comb_w)
        gates = (_dott(xm.astype(_BF), wih) + bih
                 + _dott(h_bf, whh) + bhh)
        i_g = gates[:, 0:16]
        f_g = gates[:, 16:32]
        g_g = gates[:, 32:48]
        o_g = gates[:, 48:64]
        c = jax.nn.sigmoid(f_g) * c + jax.nn.sigmoid(i_g) * jnp.tanh(g_g)
        h = jax.nn.sigmoid(o_g) * jnp.tanh(c)

    out_ref[...] = _dott(h.astype(_BF), wd) + bd


def _mlp_weights(p):
    out = []
    for i in range(4):
        out.append(p[f"W{i}"])
        out.append(p[f"b{i}"].reshape(1, -1))
    return out


def kernel(X, params):
    Xf = X.reshape(B * NV, 9).astype(jnp.float32)
    rc = jnp.asarray(np.tile(_RC, (B, 1)))
    xin = jnp.concatenate([rc, Xf], axis=1)

    weights = (
        _mlp_weights(params["inp_enc"])
        + _mlp_weights(params["msg_comb"])
        + _mlp_weights(params["msg_enc"])
        + [params["W_ih"], params["b_ih"].reshape(1, -1),
           params["W_hh"], params["b_hh"].reshape(1, -1),
           params["Wd"], params["bd"].reshape(1, -1),
           jnp.asarray(_G2, _BF)]
    )

    grid = (B // TB,)
    in_specs = [pl.BlockSpec((TB * NV, 25), lambda i: (i, 0))]
    for a in weights:
        in_specs.append(pl.BlockSpec(a.shape, lambda i: (0,) * a.ndim))

    out = pl.pallas_call(
        _body,
        grid=grid,
        in_specs=in_specs,
        out_specs=pl.BlockSpec((TB * NV, 8), lambda i: (i, 0)),
        out_shape=jax.ShapeDtypeStruct((B * NV, 8), jnp.float32),
        compiler_params=pltpu.CompilerParams(
            dimension_semantics=("arbitrary",)),
    )(xin, *weights)
    return out


# speedup vs baseline: 3.8497x; 1.0086x over previous
import numpy as np
import jax
import jax.numpy as jnp
from jax.experimental import pallas as pl
from jax.experimental.pallas import tpu as pltpu

N_STEPS = 4
TB = 32
B = 256
NV = 64
NE = 1152
DEG = 18


def _build_graph():
    s = set()
    for i in range(8):
        for j in range(8):
            start = 8 * i + j
            for x in range(8):
                s.add((start, 8 * i + x))
                s.add((start, 8 * x + j))
            bx = i // 2 * 2
            by = j // 4 * 4
            for x in range(2):
                for y in range(4):
                    s.add((start, 8 * (bx + x) + (by + y)))
    pairs = sorted(s)
    l = np.array([p[0] for p in pairs], dtype=np.int32)
    r = np.array([p[1] for p in pairs], dtype=np.int32)
    return l, r


_L, _R = _build_graph()

_perm = np.argsort(np.arange(NE) % DEG, kind="stable")
_Lp, _Rp = _L[_perm], _R[_perm]

_GL = np.zeros((NE, NV), np.float32)
_GL[np.arange(NE), _Lp] = 1.0
_GR = np.zeros((NE, NV), np.float32)
_GR[np.arange(NE), _Rp] = 1.0

_G2 = np.concatenate([_GL, _GR], axis=1)

_t = np.eye(8, dtype=np.float32)
_RC = np.concatenate([np.tile(_t, (8, 1)), np.repeat(_t, 8, axis=0)], axis=1)

_BF = jnp.bfloat16


def _dot(a, b):
    return jnp.dot(a, b, preferred_element_type=jnp.float32)


def _dotb(a, b):
    return jnp.dot(a, b, preferred_element_type=jnp.float32).astype(_BF)


def _dott(a, b):
    return jax.lax.dot_general(a, b, (((1,), (1,)), ((), ())),
                               preferred_element_type=jnp.float32)


def _dottb(a, b):
    return _dott(a, b).astype(_BF)


def _mlp(x_bf, w):
    for i in range(3):
        x_bf = jnp.maximum(
            _dottb(x_bf, w[2 * i]) + w[2 * i + 1].astype(_BF), 0.0)
    return _dott(x_bf, w[6]) + w[7]


def _body(x_ref, *refs):
    out_ref = refs[-1]
    raw = [r[...] for r in refs[:-1]]
    w = [a.astype(_BF) if (a.ndim == 2 and a.shape[0] > 1) else a
         for a in raw]
    inp_w, comb_w = w[0:8], w[8:16]
    (w0m, b0m, w1m, b1m, w2m, b2m, w3m, b3r,
     wih, bih, whh, bhh, wd, bd, g2) = w[16:]
    w0l = w0m[:, :16]
    w0r = w0m[:, 16:]
    b0m = b0m.astype(_BF)
    b1m = b1m.astype(_BF)
    b2m = b2m.astype(_BF)
    b3m = b3r * np.float32(DEG)

    x = x_ref[...].astype(_BF)
    xe = _mlp(x, inp_w)
    h = xe
    c = jnp.zeros_like(h)

    for _ in range(N_STEPS):
        h_bf = h.astype(_BF)
        u = _dottb(h_bf, w0l)
        v = _dottb(h_bf, w0r)
        rs = []
        for p in range(TB // 2):
            zp = []
            for s in (2 * p, 2 * p + 1):
                uv = jnp.concatenate(
                    [u[s * NV:(s + 1) * NV], v[s * NV:(s + 1) * NV]], axis=0)
                zp.append(_dotb(g2, uv))
            z = jnp.concatenate(zp, axis=0)
            z = jnp.maximum(z + b0m, 0.0)
            z = jnp.maximum(_dottb(z, w1m) + b1m, 0.0)
            z = jnp.maximum(_dottb(z, w2m) + b2m, 0.0)
            for si in range(2):
                zs = z[si * NE:(si + 1) * NE]
                blocks = [zs[k * NV:(k + 1) * NV] for k in range(DEG)]
                while len(blocks) > 1:
                    nb = [blocks[j] + blocks[j + 1]
                          for j in range(0, len(blocks) - 1, 2)]
                    if len(blocks) % 2:
                        nb.append(blocks[-1])
                    blocks = nb
                rs.append(blocks[0])
        red = jnp.concatenate(rs, axis=0)
        agg = _dott(red, w3m) + b3m
        xm = _mlp(jnp.concatenate([xe, agg], axis=1).astype(_BF), comb_w)
        gates = (_dott(xm.astype(_BF), wih) + bih
                 + _dott(h_bf, whh) + bhh)
        i_g = gates[:, 0:16]
        f_g = gates[:, 16:32]
        g_g = gates[:, 32:48]
        o_g = gates[:, 48:64]
        c = jax.nn.sigmoid(f_g) * c + jax.nn.sigmoid(i_g) * jnp.tanh(g_g)
        h = jax.nn.sigmoid(o_g) * jnp.tanh(c)

    out_ref[...] = _dott(h.astype(_BF), wd) + bd


def _mlp_weights(p):
    out = []
    for i in range(4):
        out.append(p[f"W{i}"])
        out.append(p[f"b{i}"].reshape(1, -1))
    return out


def kernel(X, params):
    Xf = X.reshape(B * NV, 9).astype(jnp.float32)
    rc = jnp.asarray(np.tile(_RC, (B, 1)))
    xin = jnp.concatenate([rc, Xf], axis=1)

    weights = (
        _mlp_weights(params["inp_enc"])
        + _mlp_weights(params["msg_comb"])
        + _mlp_weights(params["msg_enc"])
        + [params["W_ih"], params["b_ih"].reshape(1, -1),
           params["W_hh"], params["b_hh"].reshape(1, -1),
           params["Wd"], params["bd"].reshape(1, -1),
           jnp.asarray(_G2, _BF)]
    )

    grid = (B // TB,)
    in_specs = [pl.BlockSpec((TB * NV, 25), lambda i: (i, 0))]
    for a in weights:
        in_specs.append(pl.BlockSpec(a.shape, lambda i: (0,) * a.ndim))

    out = pl.pallas_call(
        _body,
        grid=grid,
        in_specs=in_specs,
        out_specs=pl.BlockSpec((TB * NV, 8), lambda i: (i, 0)),
        out_shape=jax.ShapeDtypeStruct((B * NV, 8), jnp.float32),
        compiler_params=pltpu.CompilerParams(
            dimension_semantics=("parallel",)),
    )(xin, *weights)
    return out
